# baseline (speedup 1.0000x reference)
import jax
import jax.numpy as jnp
from jax.experimental import pallas as pl
from jax.experimental.pallas import tpu as pltpu

_BM = 256
_NSPLIT = 4
_SUB = _BM // _NSPLIT


def _router_block(*refs):
    x_refs = refs[:_NSPLIT]
    w_ref, noise_ref, o_ref = refs[_NSPLIT:]
    wb = w_ref[...].astype(jnp.bfloat16)
    dims = (((1,), (1,)), ((), ()))
    for s in range(_NSPLIT):
        acc = jax.lax.dot_general(
            x_refs[s][...].astype(jnp.bfloat16), wb, dimension_numbers=dims,
            preferred_element_type=jnp.float32,
        )
        o_ref[s * _SUB:(s + 1) * _SUB, :] = (
            acc + noise_ref[s * _SUB:(s + 1) * _SUB, :]
        )


def _x_spec(s):
    return pl.BlockSpec((_SUB, 4096), lambda i, s=s: (_NSPLIT * i + s, 0))


@jax.jit
def kernel(x, W, noise):
    tokens, d_model = x.shape
    n_experts = W.shape[0]
    grid = (tokens // _BM,)
    return pl.pallas_call(
        _router_block,
        grid=grid,
        in_specs=[_x_spec(s) for s in range(_NSPLIT)] + [
            pl.BlockSpec((n_experts, d_model), lambda i: (0, 0)),
            pl.BlockSpec((_BM, n_experts), lambda i: (i, 0)),
        ],
        out_specs=pl.BlockSpec((_BM, n_experts), lambda i: (i, 0)),
        out_shape=jax.ShapeDtypeStruct((tokens, n_experts), jnp.float32),
        compiler_params=pltpu.CompilerParams(
            dimension_semantics=("arbitrary",),
            skip_device_barrier=True,
        ),
    )(*([x] * _NSPLIT), W, noise)


# speedup vs baseline: 1.1226x; 1.1226x over previous
import jax
import jax.numpy as jnp
from jax.experimental import pallas as pl
from jax.experimental.pallas import tpu as pltpu

_BM = 512
_NSPLIT = 4
_SUB = _BM // _NSPLIT


def _router_block(*refs):
    x_refs = refs[:_NSPLIT]
    w_ref, noise_ref, o_ref = refs[_NSPLIT:]
    i = pl.program_id(0)
    wb = w_ref[...].astype(jnp.bfloat16)
    dims = (((1,), (1,)), ((), ()))
    for s in range(_NSPLIT):
        acc = jax.lax.dot_general(
            x_refs[s][...].astype(jnp.bfloat16), wb, dimension_numbers=dims,
            preferred_element_type=jnp.float32,
        )
        row = i * _BM + s * _SUB
        o_ref[pl.ds(row, _SUB), :] = acc + noise_ref[pl.ds(row, _SUB), :]


def _x_spec(s):
    return pl.BlockSpec((_SUB, 4096), lambda i, s=s: (_NSPLIT * i + s, 0))


@jax.jit
def kernel(x, W, noise):
    tokens, d_model = x.shape
    n_experts = W.shape[0]
    grid = (tokens // _BM,)
    return pl.pallas_call(
        _router_block,
        grid=grid,
        in_specs=[_x_spec(s) for s in range(_NSPLIT)] + [
            pl.BlockSpec((n_experts, d_model), lambda i: (0, 0)),
            pl.BlockSpec((tokens, n_experts), lambda i: (0, 0)),
        ],
        out_specs=pl.BlockSpec((tokens, n_experts), lambda i: (0, 0)),
        out_shape=jax.ShapeDtypeStruct((tokens, n_experts), jnp.float32),
        compiler_params=pltpu.CompilerParams(
            dimension_semantics=("arbitrary",),
        ),
    )(*([x] * _NSPLIT), W, noise)


# speedup vs baseline: 1.1516x; 1.0258x over previous
import jax
import jax.numpy as jnp
from jax.experimental import pallas as pl
from jax.experimental.pallas import tpu as pltpu

_BM = 512
_NSPLIT = 4
_SUB = _BM // _NSPLIT


def _router_block(*refs):
    x_refs = refs[:_NSPLIT]
    w_ref, noise_ref, o_ref = refs[_NSPLIT:]
    wb = w_ref[...].astype(jnp.bfloat16)
    dims = (((1,), (1,)), ((), ()))
    for s in range(_NSPLIT):
        acc = jax.lax.dot_general(
            x_refs[s][...].astype(jnp.bfloat16), wb, dimension_numbers=dims,
            preferred_element_type=jnp.float32,
        )
        o_ref[s * _SUB:(s + 1) * _SUB, :] = (
            acc + noise_ref[s * _SUB:(s + 1) * _SUB, :]
        )


def _x_spec(s):
    return pl.BlockSpec((_SUB, 4096), lambda i, s=s: (_NSPLIT * i + s, 0))


@jax.jit
def kernel(x, W, noise):
    tokens, d_model = x.shape
    n_experts = W.shape[0]
    grid = (tokens // _BM,)
    return pl.pallas_call(
        _router_block,
        grid=grid,
        in_specs=[_x_spec(s) for s in range(_NSPLIT)] + [
            pl.BlockSpec((n_experts, d_model), lambda i: (0, 0)),
            pl.BlockSpec((_BM, n_experts), lambda i: (i, 0)),
        ],
        out_specs=pl.BlockSpec((_BM, n_experts), lambda i: (i, 0)),
        out_shape=jax.ShapeDtypeStruct((tokens, n_experts), jnp.float32),
        compiler_params=pltpu.CompilerParams(
            dimension_semantics=("arbitrary",),
        ),
    )(*([x] * _NSPLIT), W, noise)


# speedup vs baseline: 1.1524x; 1.0007x over previous
import jax
import jax.numpy as jnp
from jax.experimental import pallas as pl
from jax.experimental.pallas import tpu as pltpu

_BM = 512
_NSPLIT = 4
_SUB = _BM // _NSPLIT


def _router_block(*refs):
    x_refs = refs[:_NSPLIT]
    w_ref, noise_ref, o_ref = refs[_NSPLIT:]
    wb = w_ref[...].astype(jnp.bfloat16)
    dims = (((1,), (1,)), ((), ()))
    for s in range(_NSPLIT):
        acc = jax.lax.dot_general(
            x_refs[s][...].astype(jnp.bfloat16), wb, dimension_numbers=dims,
            preferred_element_type=jnp.float32,
        )
        o_ref[s * _SUB:(s + 1) * _SUB, :] = (
            acc + noise_ref[s * _SUB:(s + 1) * _SUB, :]
        )


def _x_spec(s):
    return pl.BlockSpec((_SUB, 4096), lambda i, s=s: (_NSPLIT * i + s, 0))


@jax.jit
def kernel(x, W, noise):
    tokens, d_model = x.shape
    n_experts = W.shape[0]
    grid = (tokens // _BM,)
    return pl.pallas_call(
        _router_block,
        grid=grid,
        in_specs=[_x_spec(s) for s in range(_NSPLIT)] + [
            pl.BlockSpec((n_experts, d_model), lambda i: (0, 0)),
            pl.BlockSpec((_BM, n_experts), lambda i: (i, 0)),
        ],
        out_specs=pl.BlockSpec((_BM, n_experts), lambda i: (i, 0)),
        out_shape=jax.ShapeDtypeStruct((tokens, n_experts), jnp.float32),
        compiler_params=pltpu.CompilerParams(
            dimension_semantics=("arbitrary",),
            fuse_transposed_lhs_in_matmul=True,
        ),
    )(*([x] * _NSPLIT), W, noise)
